# baseline (speedup 1.0000x reference)
import functools
import math

import jax
import jax.numpy as jnp
from jax import lax
from jax.experimental import pallas as pl
from jax.experimental.pallas import tpu as pltpu
from jax.experimental.pallas import tpu_sc as plsc

_N = 16384
_F = 26
_NQ = 256
_TOTAL = _N * _F
_NW = 32
_CHUNK = _TOTAL // _NW
_VREGS = _CHUNK // 16

_SQRT2 = math.sqrt(2.0)
_LN2 = 0.6931471805599453
_C_CENTRAL = tuple(c * _SQRT2 for c in (
    2.81022636e-08, 3.43273939e-07, -3.5233877e-06, -4.39150654e-06,
    0.00021858087, -0.00125372503, -0.00417768164, 0.246640727, 1.50140941))
_C_TAIL = tuple(c * _SQRT2 for c in (
    -0.000200214257, 0.000100950558, 0.00134934322, -0.00367342844,
    0.00573950773, -0.0076224613, 0.00943887047, 1.00167406, 2.83297682))
_C_NEGLOG = (1.720806112e-02, -8.172680837e-02, 1.887826736e-01,
             -3.145905354e-01, 4.969779112e-01, -9.997924357e-01,
             -3.507552053e-06)


def _horner(coeffs, v):
    p = jnp.full((16,), coeffs[0], dtype=jnp.float32)
    for c in coeffs[1:]:
        p = p * v + jnp.float32(c)
    return p


_mesh = plsc.VectorSubcoreMesh(core_axis_name="c", subcore_axis_name="s")


@functools.partial(
    pl.kernel,
    out_type=jax.ShapeDtypeStruct((_TOTAL,), jnp.float32),
    mesh=_mesh,
    scratch_types=[
        pltpu.VMEM((_NQ * _F,), jnp.float32),
        pltpu.VMEM((_CHUNK,), jnp.float32),
        pltpu.VMEM((_CHUNK,), jnp.float32),
    ],
    compiler_params=pltpu.CompilerParams(needs_layout_passes=False),
)
def _qp_sc(x_hbm, qt_hbm, out_hbm, q_v, x_v, y_v):
    cid = lax.axis_index("c")
    sid = lax.axis_index("s")
    wid = sid * 2 + cid
    base = wid * _CHUNK
    pltpu.sync_copy(qt_hbm, q_v)
    pltpu.sync_copy(x_hbm.at[pl.ds(base, _CHUNK)], x_v)

    lanes0 = lax.iota(jnp.int32, 16)

    @plsc.parallel_loop(0, _VREGS, unroll=8)
    def _body(i):
        off = i * 16
        xv = x_v[pl.ds(off, 16)]
        feat = lax.rem(base + off + lanes0, _F)
        fb1 = feat * _NQ - 1

        gpos = fb1
        for step in (128, 64, 32, 16, 8, 4, 2, 1):
            cand = gpos + step
            v = plsc.load_gather(q_v, [cand])
            gpos = jnp.where(v < xv, cand, gpos)

        g0 = jnp.clip(gpos, fb1 + 1, fb1 + 255)
        idxf = (g0 - (fb1 + 1)).astype(jnp.float32)
        last = plsc.load_gather(q_v, [g0])
        nxt = plsc.load_gather(q_v, [g0 + 1])
        diff = nxt - last
        dz = diff == 0.0
        safe = jnp.where(dz, jnp.float32(1.0), diff)
        interp = jnp.where(dz, jnp.float32(0.5), (xv - last) / safe)
        y = jnp.clip((idxf + interp) * jnp.float32(1.0 / _NQ), 0.0, 1.0)

        ys = jnp.clip(y, jnp.float32(1e-6), jnp.float32(1.0 - 1e-6))
        t = jnp.float32(2.0) * ys - jnp.float32(1.0)
        u = (jnp.float32(1.0) - t) * (jnp.float32(1.0) + t)

        bits = plsc.bitcast(u, jnp.int32)
        e = (bits >> 23) - 127
        z = plsc.bitcast((bits & 0x007FFFFF) | 0x3F800000, jnp.float32) - \
            jnp.float32(1.0)
        w = e.astype(jnp.float32) * jnp.float32(-_LN2) + _horner(_C_NEGLOG, z)

        p1 = _horner(_C_CENTRAL, w - jnp.float32(2.5))

        wp = jnp.maximum(w, jnp.float32(1e-10))
        rs = plsc.bitcast(0x5F3759DF - (plsc.bitcast(wp, jnp.int32) >> 1),
                          jnp.float32)
        rs = rs * (jnp.float32(1.5) - jnp.float32(0.5) * wp * rs * rs)
        rs = rs * (jnp.float32(1.5) - jnp.float32(0.5) * wp * rs * rs)
        s = wp * rs
        p2 = _horner(_C_TAIL, s - jnp.float32(3.0))

        p = jnp.where(w < jnp.float32(5.0), p1, p2)
        g = p * t
        out = jnp.where(y <= jnp.float32(0.0), jnp.float32(-100.0),
                        jnp.where(y >= jnp.float32(1.0), jnp.float32(100.0), g))
        y_v[pl.ds(off, 16)] = out

    pltpu.sync_copy(y_v, out_hbm.at[pl.ds(base, _CHUNK)])


def kernel(x, quantiles):
    xf = x.reshape(-1)
    qt = quantiles.T.reshape(-1)
    yf = _qp_sc(xf, qt)
    return yf.reshape(x.shape)


# speedup vs baseline: 1.4849x; 1.4849x over previous
import functools
import math

import jax
import jax.numpy as jnp
from jax import lax
from jax.experimental import pallas as pl
from jax.experimental.pallas import tpu as pltpu
from jax.experimental.pallas import tpu_sc as plsc

_N = 16384
_F = 26
_NQ = 256
_TOTAL = _N * _F
_NW = 32
_CHUNK = _TOTAL // _NW
_VREGS = _CHUNK // 16
_STRIDE = _NQ + 1

_SQRT2 = math.sqrt(2.0)
_LN2 = 0.6931471805599453
_C_CENTRAL = tuple(c * _SQRT2 for c in (
    2.81022636e-08, 3.43273939e-07, -3.5233877e-06, -4.39150654e-06,
    0.00021858087, -0.00125372503, -0.00417768164, 0.246640727, 1.50140941))
_C_TAIL = tuple(c * _SQRT2 for c in (
    -0.000200214257, 0.000100950558, 0.00134934322, -0.00367342844,
    0.00573950773, -0.0076224613, 0.00943887047, 1.00167406, 2.83297682))
_C_NEGLOG = (1.720806112e-02, -8.172680837e-02, 1.887826736e-01,
             -3.145905354e-01, 4.969779112e-01, -9.997924357e-01,
             -3.507552053e-06)


def _horner(coeffs, v):
    p = jnp.full((16,), coeffs[0], dtype=jnp.float32)
    for c in coeffs[1:]:
        p = p * v + jnp.float32(c)
    return p


_mesh = plsc.VectorSubcoreMesh(core_axis_name="c", subcore_axis_name="s")


@functools.partial(
    pl.kernel,
    out_type=jax.ShapeDtypeStruct((_TOTAL,), jnp.float32),
    mesh=_mesh,
    scratch_types=[
        pltpu.VMEM((_F * _STRIDE,), jnp.float32),
        pltpu.VMEM((_CHUNK,), jnp.float32),
        pltpu.VMEM((_CHUNK,), jnp.float32),
    ],
    compiler_params=pltpu.CompilerParams(needs_layout_passes=False),
)
def _qp_sc(x_hbm, qt_hbm, out_hbm, q_v, x_v, y_v):
    cid = lax.axis_index("c")
    sid = lax.axis_index("s")
    wid = sid * 2 + cid
    base = wid * _CHUNK
    pltpu.sync_copy(qt_hbm, q_v)
    pltpu.sync_copy(x_hbm.at[pl.ds(base, _CHUNK)], x_v)

    lanes0 = lax.iota(jnp.int32, 16)

    @plsc.parallel_loop(0, _VREGS, unroll=4)
    def _body(i):
        off = i * 16
        xv = x_v[pl.ds(off, 16)]
        feat = lax.rem(base + off + lanes0, _F)
        fb1 = feat * _STRIDE - 1

        gpos = fb1
        for step in (128, 64, 32, 16, 8, 4, 2, 1):
            cand = gpos + step
            v = plsc.load_gather(q_v, [cand])
            gpos = jnp.where(v < xv, cand, gpos)

        g0 = jnp.clip(gpos, fb1 + 1, fb1 + 255)
        idxf = (g0 - (fb1 + 1)).astype(jnp.float32)
        last = plsc.load_gather(q_v, [g0])
        nxt = plsc.load_gather(q_v, [g0 + 1])
        diff = nxt - last
        dz = diff == 0.0
        safe = jnp.where(dz, jnp.float32(1.0), diff)
        interp = jnp.where(dz, jnp.float32(0.5), (xv - last) / safe)
        y = jnp.clip((idxf + interp) * jnp.float32(1.0 / _NQ), 0.0, 1.0)

        ys = jnp.clip(y, jnp.float32(1e-6), jnp.float32(1.0 - 1e-6))
        t = jnp.float32(2.0) * ys - jnp.float32(1.0)
        u = (jnp.float32(1.0) - t) * (jnp.float32(1.0) + t)

        bits = plsc.bitcast(u, jnp.int32)
        e = (bits >> 23) - 127
        z = plsc.bitcast((bits & 0x007FFFFF) | 0x3F800000, jnp.float32) - \
            jnp.float32(1.0)
        w = e.astype(jnp.float32) * jnp.float32(-_LN2) + _horner(_C_NEGLOG, z)

        p1 = _horner(_C_CENTRAL, w - jnp.float32(2.5))

        wp = jnp.maximum(w, jnp.float32(1e-10))
        rs = plsc.bitcast(0x5F3759DF - (plsc.bitcast(wp, jnp.int32) >> 1),
                          jnp.float32)
        rs = rs * (jnp.float32(1.5) - jnp.float32(0.5) * wp * rs * rs)
        rs = rs * (jnp.float32(1.5) - jnp.float32(0.5) * wp * rs * rs)
        s = wp * rs
        p2 = _horner(_C_TAIL, s - jnp.float32(3.0))

        p = jnp.where(w < jnp.float32(5.0), p1, p2)
        g = p * t
        out = jnp.where(y <= jnp.float32(0.0), jnp.float32(-100.0),
                        jnp.where(y >= jnp.float32(1.0), jnp.float32(100.0), g))
        y_v[pl.ds(off, 16)] = out

    pltpu.sync_copy(y_v, out_hbm.at[pl.ds(base, _CHUNK)])


def kernel(x, quantiles):
    xf = x.reshape(-1)
    qt = jnp.pad(quantiles.T, ((0, 0), (0, 1))).reshape(-1)
    yf = _qp_sc(xf, qt)
    return yf.reshape(x.shape)


# speedup vs baseline: 1.8457x; 1.2430x over previous
import functools
import math

import jax
import jax.numpy as jnp
from jax import lax
from jax.experimental import pallas as pl
from jax.experimental.pallas import tpu as pltpu
from jax.experimental.pallas import tpu_sc as plsc

_N = 16384
_F = 26
_NQ = 256
_TOTAL = _N * _F
_NW = 32
_CHUNK = _TOTAL // _NW
_VREGS = _CHUNK // 16
_ROWS = _N // _NW
_STRIDE = _NQ + 1

_SQRT2 = math.sqrt(2.0)
_LN2 = 0.6931471805599453
_C_CENTRAL = tuple(c * _SQRT2 for c in (
    2.81022636e-08, 3.43273939e-07, -3.5233877e-06, -4.39150654e-06,
    0.00021858087, -0.00125372503, -0.00417768164, 0.246640727, 1.50140941))
_C_TAIL = tuple(c * _SQRT2 for c in (
    -0.000200214257, 0.000100950558, 0.00134934322, -0.00367342844,
    0.00573950773, -0.0076224613, 0.00943887047, 1.00167406, 2.83297682))
_C_NEGLOG = (1.720806112e-02, -8.172680837e-02, 1.887826736e-01,
             -3.145905354e-01, 4.969779112e-01, -9.997924357e-01,
             -3.507552053e-06)


def _horner(coeffs, v):
    p = jnp.full((16,), coeffs[0], dtype=jnp.float32)
    for c in coeffs[1:]:
        p = p * v + jnp.float32(c)
    return p


_mesh = plsc.VectorSubcoreMesh(core_axis_name="c", subcore_axis_name="s")


@functools.partial(
    pl.kernel,
    out_type=jax.ShapeDtypeStruct((_N, _F), jnp.float32),
    mesh=_mesh,
    scratch_types=[
        pltpu.VMEM((_F * _STRIDE,), jnp.float32),
        pltpu.VMEM((_ROWS, _F), jnp.float32),
    ],
    compiler_params=pltpu.CompilerParams(needs_layout_passes=False),
)
def _qp_sc(x_hbm, qt_hbm, out_hbm, q_v, x_v):
    cid = lax.axis_index("c")
    sid = lax.axis_index("s")
    wid = sid * 2 + cid
    r0 = wid * _ROWS
    pltpu.sync_copy(qt_hbm, q_v)
    pltpu.sync_copy(x_hbm.at[pl.ds(r0, _ROWS), :], x_v)

    lanes0 = lax.iota(jnp.int32, 16)

    @plsc.parallel_loop(0, _VREGS, unroll=4)
    def _body(i):
        kk = i * 16 + lanes0
        lrow = kk // _F
        feat = kk - lrow * _F
        xv = plsc.load_gather(x_v, [lrow, feat])
        fb1 = feat * _STRIDE - 1

        gpos = fb1
        for step in (128, 64, 32, 16, 8, 4, 2, 1):
            cand = gpos + step
            v = plsc.load_gather(q_v, [cand])
            gpos = jnp.where(v < xv, cand, gpos)

        g0 = jnp.clip(gpos, fb1 + 1, fb1 + 255)
        idxf = (g0 - (fb1 + 1)).astype(jnp.float32)
        last = plsc.load_gather(q_v, [g0])
        nxt = plsc.load_gather(q_v, [g0 + 1])
        diff = nxt - last
        dz = diff == 0.0
        safe = jnp.where(dz, jnp.float32(1.0), diff)
        interp = jnp.where(dz, jnp.float32(0.5), (xv - last) / safe)
        y = jnp.clip((idxf + interp) * jnp.float32(1.0 / _NQ), 0.0, 1.0)

        ys = jnp.clip(y, jnp.float32(1e-6), jnp.float32(1.0 - 1e-6))
        t = jnp.float32(2.0) * ys - jnp.float32(1.0)
        u = (jnp.float32(1.0) - t) * (jnp.float32(1.0) + t)

        bits = plsc.bitcast(u, jnp.int32)
        e = (bits >> 23) - 127
        z = plsc.bitcast((bits & 0x007FFFFF) | 0x3F800000, jnp.float32) - \
            jnp.float32(1.0)
        w = e.astype(jnp.float32) * jnp.float32(-_LN2) + _horner(_C_NEGLOG, z)

        p1 = _horner(_C_CENTRAL, w - jnp.float32(2.5))

        wp = jnp.maximum(w, jnp.float32(1e-10))
        rs = plsc.bitcast(0x5F3759DF - (plsc.bitcast(wp, jnp.int32) >> 1),
                          jnp.float32)
        rs = rs * (jnp.float32(1.5) - jnp.float32(0.5) * wp * rs * rs)
        rs = rs * (jnp.float32(1.5) - jnp.float32(0.5) * wp * rs * rs)
        s = wp * rs
        p2 = _horner(_C_TAIL, s - jnp.float32(3.0))

        p = jnp.where(w < jnp.float32(5.0), p1, p2)
        g = p * t
        out = jnp.where(y <= jnp.float32(0.0), jnp.float32(-100.0),
                        jnp.where(y >= jnp.float32(1.0), jnp.float32(100.0), g))
        plsc.store_scatter(x_v, [lrow, feat], out)

    pltpu.sync_copy(x_v, out_hbm.at[pl.ds(r0, _ROWS), :])


def kernel(x, quantiles):
    qt = jnp.pad(quantiles.T, ((0, 0), (0, 1))).reshape(-1)
    return _qp_sc(x, qt)


# speedup vs baseline: 2.0485x; 1.1099x over previous
import functools
import math

import jax
import jax.numpy as jnp
from jax import lax
from jax.experimental import pallas as pl
from jax.experimental.pallas import tpu as pltpu
from jax.experimental.pallas import tpu_sc as plsc

_N = 16384
_F = 26
_NQ = 256
_TOTAL = _N * _F
_NW = 32
_CHUNK = _TOTAL // _NW
_VREGS = _CHUNK // 16
_COLS = _N // _NW
_STRIDE = _NQ + 1

_SQRT2 = math.sqrt(2.0)
_LN2 = 0.6931471805599453
_C_CENTRAL = tuple(c * _SQRT2 for c in (
    2.81022636e-08, 3.43273939e-07, -3.5233877e-06, -4.39150654e-06,
    0.00021858087, -0.00125372503, -0.00417768164, 0.246640727, 1.50140941))
_C_TAIL = tuple(c * _SQRT2 for c in (
    -0.000200214257, 0.000100950558, 0.00134934322, -0.00367342844,
    0.00573950773, -0.0076224613, 0.00943887047, 1.00167406, 2.83297682))
_C_NEGLOG = (1.720806112e-02, -8.172680837e-02, 1.887826736e-01,
             -3.145905354e-01, 4.969779112e-01, -9.997924357e-01,
             -3.507552053e-06)


def _horner(coeffs, v):
    p = jnp.full((16,), coeffs[0], dtype=jnp.float32)
    for c in coeffs[1:]:
        p = p * v + jnp.float32(c)
    return p


_mesh = plsc.VectorSubcoreMesh(core_axis_name="c", subcore_axis_name="s")


@functools.partial(
    pl.kernel,
    out_type=jax.ShapeDtypeStruct((_F, _N), jnp.float32),
    mesh=_mesh,
    scratch_types=[
        pltpu.VMEM((_F * _STRIDE,), jnp.float32),
        pltpu.VMEM((_F, _COLS), jnp.float32),
    ],
    compiler_params=pltpu.CompilerParams(needs_layout_passes=False),
)
def _qp_sc(x_hbm, qt_hbm, out_hbm, q_v, x_v):
    cid = lax.axis_index("c")
    sid = lax.axis_index("s")
    wid = sid * 2 + cid
    c0 = wid * _COLS
    pltpu.sync_copy(qt_hbm, q_v)
    pltpu.sync_copy(x_hbm.at[:, pl.ds(c0, _COLS)], x_v)

    lanes0 = lax.iota(jnp.int32, 16)

    @plsc.parallel_loop(0, _VREGS, unroll=4)
    def _body(i):
        kk = i * 16 + lanes0
        col = kk // _F
        feat = kk - col * _F
        xv = plsc.load_gather(x_v, [feat, col])
        fb1 = feat * _STRIDE - 1

        gpos = fb1
        for step in (128, 64, 32, 16, 8, 4, 2, 1):
            cand = gpos + step
            v = plsc.load_gather(q_v, [cand])
            gpos = jnp.where(v < xv, cand, gpos)

        g0 = jnp.clip(gpos, fb1 + 1, fb1 + 255)
        idxf = (g0 - (fb1 + 1)).astype(jnp.float32)
        last = plsc.load_gather(q_v, [g0])
        nxt = plsc.load_gather(q_v, [g0 + 1])
        diff = nxt - last
        dz = diff == 0.0
        safe = jnp.where(dz, jnp.float32(1.0), diff)
        interp = jnp.where(dz, jnp.float32(0.5), (xv - last) / safe)
        y = jnp.clip((idxf + interp) * jnp.float32(1.0 / _NQ), 0.0, 1.0)

        ys = jnp.clip(y, jnp.float32(1e-6), jnp.float32(1.0 - 1e-6))
        t = jnp.float32(2.0) * ys - jnp.float32(1.0)
        u = (jnp.float32(1.0) - t) * (jnp.float32(1.0) + t)

        bits = plsc.bitcast(u, jnp.int32)
        e = (bits >> 23) - 127
        z = plsc.bitcast((bits & 0x007FFFFF) | 0x3F800000, jnp.float32) - \
            jnp.float32(1.0)
        w = e.astype(jnp.float32) * jnp.float32(-_LN2) + _horner(_C_NEGLOG, z)

        p1 = _horner(_C_CENTRAL, w - jnp.float32(2.5))

        wp = jnp.maximum(w, jnp.float32(1e-10))
        rs = plsc.bitcast(0x5F3759DF - (plsc.bitcast(wp, jnp.int32) >> 1),
                          jnp.float32)
        rs = rs * (jnp.float32(1.5) - jnp.float32(0.5) * wp * rs * rs)
        rs = rs * (jnp.float32(1.5) - jnp.float32(0.5) * wp * rs * rs)
        s = wp * rs
        p2 = _horner(_C_TAIL, s - jnp.float32(3.0))

        p = jnp.where(w < jnp.float32(5.0), p1, p2)
        g = p * t
        out = jnp.where(y <= jnp.float32(0.0), jnp.float32(-100.0),
                        jnp.where(y >= jnp.float32(1.0), jnp.float32(100.0), g))
        plsc.store_scatter(x_v, [feat, col], out)

    pltpu.sync_copy(x_v, out_hbm.at[:, pl.ds(c0, _COLS)])


def kernel(x, quantiles):
    qt = jnp.pad(quantiles.T, ((0, 0), (0, 1))).reshape(-1)
    return _qp_sc(x.T, qt).T


# speedup vs baseline: 2.4208x; 1.1817x over previous
import functools
import math

import jax
import jax.numpy as jnp
from jax import lax
from jax.experimental import pallas as pl
from jax.experimental.pallas import tpu as pltpu
from jax.experimental.pallas import tpu_sc as plsc

_N = 16384
_F = 26
_NQ = 256
_TOTAL = _N * _F
_NW = 32
_CHUNK = _TOTAL // _NW
_VREGS = _CHUNK // 16
_COLS = _N // _NW
_STRIDE = _NQ + 1

_SQRT2 = math.sqrt(2.0)
_LN2 = 0.6931471805599453
_C_CENTRAL = tuple(c * _SQRT2 for c in (
    2.81022636e-08, 3.43273939e-07, -3.5233877e-06, -4.39150654e-06,
    0.00021858087, -0.00125372503, -0.00417768164, 0.246640727, 1.50140941))
_C_TAIL = tuple(c * _SQRT2 for c in (
    -0.000200214257, 0.000100950558, 0.00134934322, -0.00367342844,
    0.00573950773, -0.0076224613, 0.00943887047, 1.00167406, 2.83297682))
_C_NEGLOG = (1.720806112e-02, -8.172680837e-02, 1.887826736e-01,
             -3.145905354e-01, 4.969779112e-01, -9.997924357e-01,
             -3.507552053e-06)


def _horner(coeffs, v):
    p = jnp.full((16,), coeffs[0], dtype=jnp.float32)
    for c in coeffs[1:]:
        p = p * v + jnp.float32(c)
    return p


_mesh = plsc.VectorSubcoreMesh(core_axis_name="c", subcore_axis_name="s")


@functools.partial(
    pl.kernel,
    out_type=jax.ShapeDtypeStruct((_F, _N), jnp.float32),
    mesh=_mesh,
    scratch_types=[
        pltpu.VMEM((_F * _STRIDE,), jnp.float32),
        pltpu.VMEM((_F, _COLS), jnp.float32),
    ],
    compiler_params=pltpu.CompilerParams(needs_layout_passes=False),
)
def _qp_sc(x_hbm, qt_hbm, out_hbm, q_v, x_v):
    cid = lax.axis_index("c")
    sid = lax.axis_index("s")
    wid = sid * 2 + cid
    c0 = wid * _COLS
    pltpu.sync_copy(qt_hbm, q_v)
    pltpu.sync_copy(x_hbm.at[:, pl.ds(c0, _COLS)], x_v)

    lanes0 = lax.iota(jnp.int32, 16)

    @plsc.parallel_loop(0, _VREGS, unroll=4)
    def _body(i):
        kk = i * 16 + lanes0
        colb = kk // _F
        feat = kk - colb * _F
        col = (colb + feat) & (_COLS - 1)
        xv = plsc.load_gather(x_v, [feat, col])
        fb1 = feat * _STRIDE - 1

        gpos = fb1
        for step in (128, 64, 32, 16, 8, 4, 2, 1):
            cand = gpos + step
            v = plsc.load_gather(q_v, [cand])
            gpos = jnp.where(v < xv, cand, gpos)

        g0 = jnp.clip(gpos, fb1 + 1, fb1 + 255)
        idxf = (g0 - (fb1 + 1)).astype(jnp.float32)
        last = plsc.load_gather(q_v, [g0])
        nxt = plsc.load_gather(q_v, [g0 + 1])
        diff = nxt - last
        dz = diff == 0.0
        safe = jnp.where(dz, jnp.float32(1.0), diff)
        interp = jnp.where(dz, jnp.float32(0.5), (xv - last) / safe)
        y = jnp.clip((idxf + interp) * jnp.float32(1.0 / _NQ), 0.0, 1.0)

        ys = jnp.clip(y, jnp.float32(1e-6), jnp.float32(1.0 - 1e-6))
        t = jnp.float32(2.0) * ys - jnp.float32(1.0)
        u = (jnp.float32(1.0) - t) * (jnp.float32(1.0) + t)

        bits = plsc.bitcast(u, jnp.int32)
        e = (bits >> 23) - 127
        z = plsc.bitcast((bits & 0x007FFFFF) | 0x3F800000, jnp.float32) - \
            jnp.float32(1.0)
        w = e.astype(jnp.float32) * jnp.float32(-_LN2) + _horner(_C_NEGLOG, z)

        p1 = _horner(_C_CENTRAL, w - jnp.float32(2.5))

        wp = jnp.maximum(w, jnp.float32(1e-10))
        rs = plsc.bitcast(0x5F3759DF - (plsc.bitcast(wp, jnp.int32) >> 1),
                          jnp.float32)
        rs = rs * (jnp.float32(1.5) - jnp.float32(0.5) * wp * rs * rs)
        rs = rs * (jnp.float32(1.5) - jnp.float32(0.5) * wp * rs * rs)
        s = wp * rs
        p2 = _horner(_C_TAIL, s - jnp.float32(3.0))

        p = jnp.where(w < jnp.float32(5.0), p1, p2)
        g = p * t
        out = jnp.where(y <= jnp.float32(0.0), jnp.float32(-100.0),
                        jnp.where(y >= jnp.float32(1.0), jnp.float32(100.0), g))
        plsc.store_scatter(x_v, [feat, col], out)

    pltpu.sync_copy(x_v, out_hbm.at[:, pl.ds(c0, _COLS)])


def kernel(x, quantiles):
    qt = jnp.pad(quantiles.T, ((0, 0), (0, 1))).reshape(-1)
    return _qp_sc(x.T, qt).T
